# baseline (speedup 1.0000x reference)
import functools

import jax
import jax.numpy as jnp
from jax import lax
from jax.experimental import pallas as pl
from jax.experimental.pallas import tpu as pltpu
from jax.experimental.pallas import tpu_sc as plsc

_EPS = 0.5
_N = 4096
_NC = 2
_NS = 16
_NW = _NC * _NS
_ROWS_PER_W = _N // _NW
_CHUNK = 4
_NCHUNK = _ROWS_PER_W // _CHUNK
_LANES = 16

_mesh = plsc.VectorSubcoreMesh(core_axis_name="c", subcore_axis_name="s")


@functools.partial(
    pl.kernel,
    out_type=jax.ShapeDtypeStruct((_N, _N), jnp.float32),
    mesh=_mesh,
    scratch_types=[
        pltpu.VMEM((_CHUNK, _N), jnp.float32),
        pltpu.VMEM((_CHUNK, _N), jnp.float32),
        pltpu.VMEM((_CHUNK, _N), jnp.float32),
        pltpu.VMEM((_CHUNK, _N), jnp.float32),
        pltpu.SemaphoreType.DMA,
        pltpu.SemaphoreType.DMA,
        pltpu.SemaphoreType.DMA,
        pltpu.SemaphoreType.DMA,
    ],
)
def _sc_mask(adj_hbm, out_hbm, ib0, ib1, ob0, ob1, si0, si1, so0, so1):
    ibufs = (ib0, ib1)
    obufs = (ob0, ob1)
    isems = (si0, si1)
    osems = (so0, so1)

    wid = lax.axis_index("s") * _NC + lax.axis_index("c")
    base = wid * _ROWS_PER_W

    def start_in(k):
        b = k & 1
        return pltpu.async_copy(
            adj_hbm.at[pl.ds(base + k * _CHUNK, _CHUNK)], ibufs[b], isems[b]
        )

    def compute(b):
        def body(j, carry):
            c0 = j * _LANES
            for r in range(_CHUNK):
                v = ibufs[b][r, pl.ds(c0, _LANES)]
                obufs[b][r, pl.ds(c0, _LANES)] = jnp.where(v > _EPS, v, 0.0)
            return carry

        lax.fori_loop(0, _N // _LANES, body, 0)

    cp_in = [start_in(0), start_in(1)]
    pending_out = [None, None]
    for k in range(_NCHUNK):
        b = k & 1
        cp_in[b].wait()
        if pending_out[b] is not None:
            pending_out[b].wait()
        compute(b)
        pending_out[b] = pltpu.async_copy(
            obufs[b], out_hbm.at[pl.ds(base + k * _CHUNK, _CHUNK)], osems[b]
        )
        if k + 2 < _NCHUNK:
            cp_in[b] = start_in(k + 2)
    pending_out[0].wait()
    pending_out[1].wait()


def kernel(adj):
    return _sc_mask(adj)
